# Initial kernel scaffold; baseline (speedup 1.0000x reference)
import jax
import jax.numpy as jnp
from jax.experimental import pallas as pl


def kernel(cart, neigh_list, shifts, species, rs, inta, params):
    raise NotImplementedError("write your pallas kernel here")


# speedup vs baseline: 15.1863x; 15.1863x over previous
import numpy as np
import jax
import jax.numpy as jnp
from jax import lax
from jax.experimental import pallas as pl
from jax.experimental.pallas import tpu as pltpu
from jax.experimental.pallas import tpu_sc as plsc

N_NODES = 10000
N_EDGES = 160000
NWAVE = 8
P = 40
NC, NS, L = 2, 16, 16
WH = NWAVE // NC
F = P * WH
EPT = N_EDGES // NS
CHUNK = 80
NCHUNK = EPT // CHUNK
GPC = CHUNK // L
N_PAD = 10240
NPT = N_PAD // NS

_TWO_PI_SQ = float(4.0 * np.pi * np.pi)
_COS_COEFS = [1.0 / 479001600.0, -1.0 / 3628800.0, 1.0 / 40320.0,
              -1.0 / 720.0, 1.0 / 24.0, -0.5, 1.0]

_INDEX_PARA = np.repeat(np.arange(4), [1, 3, 9, 27])


def _build_sel_matrix():
    m = np.zeros((2 * F, 32), np.float32)
    for j in range(2 * F):
        c, jj = divmod(j, F)
        p, wl = divmod(jj, WH)
        m[j, _INDEX_PARA[p] * NWAVE + c * WH + wl] = 1.0
    return m


_SEL_M_NP = _build_sel_matrix()


def _sc_body(ctab_h, dst_h, src_h, shx_h, shy_h, shz_h,
             rs_h, inta_h, par_h, zero_h, orb_h,
             rs_t, inta_t, par_t, dstb, srcb, shxb, shyb, shzb,
             crowsd, crowss, radb, acc):
    c = lax.axis_index("c")
    s = lax.axis_index("s")

    pltpu.sync_copy(rs_h, rs_t)
    pltpu.sync_copy(inta_h, inta_t)
    pltpu.sync_copy(par_h, par_t)

    pltpu.sync_copy(zero_h.at[pl.ds(s * NPT, NPT)], acc.at[pl.ds(s * NPT, NPT)])
    plsc.subcore_barrier()

    wbase = c * WH
    iota = lax.iota(jnp.int32, L)
    czero = jnp.full((L,), 0, jnp.int32)
    cone = jnp.full((L,), 1, jnp.int32)
    ctwo = jnp.full((L,), 2, jnp.int32)
    cthree = jnp.full((L,), 3, jnp.int32)

    def chunk_body(ci, carry):
        ebase = s * EPT + ci * CHUNK
        pltpu.sync_copy(dst_h.at[pl.ds(ebase, CHUNK)], dstb)
        pltpu.sync_copy(src_h.at[pl.ds(ebase, CHUNK)], srcb)
        pltpu.sync_copy(shx_h.at[pl.ds(ebase, CHUNK)], shxb)
        pltpu.sync_copy(shy_h.at[pl.ds(ebase, CHUNK)], shyb)
        pltpu.sync_copy(shz_h.at[pl.ds(ebase, CHUNK)], shzb)
        pltpu.sync_copy(ctab_h.at[dstb], crowsd)
        pltpu.sync_copy(ctab_h.at[srcb], crowss)
        for g in range(GPC):
            rowv = iota + (g * L)
            xi = plsc.load_gather(crowsd, [rowv, czero])
            yi = plsc.load_gather(crowsd, [rowv, cone])
            zi = plsc.load_gather(crowsd, [rowv, ctwo])
            xj = plsc.load_gather(crowss, [rowv, czero])
            yj = plsc.load_gather(crowss, [rowv, cone])
            zj = plsc.load_gather(crowss, [rowv, ctwo])
            spf = plsc.load_gather(crowss, [rowv, cthree])
            dx = xi - xj - shxb[pl.ds(g * L, L)]
            dy = yi - yj - shyb[pl.ds(g * L, L)]
            dz = zi - zj - shzb[pl.ds(g * L, L)]
            r2 = dx * dx + dy * dy + dz * dz
            bits = plsc.bitcast(r2, jnp.int32)
            bits = jnp.int32(0x5F3759DF) - (bits >> 1)
            yv = plsc.bitcast(bits, jnp.float32)
            for _ in range(3):
                yv = yv * (jnp.float32(1.5) - jnp.float32(0.5) * r2 * yv * yv)
            dist = r2 * yv
            t = dist * jnp.float32(0.1)
            frac = t - t.astype(jnp.int32).astype(jnp.float32)
            sv = frac - jnp.float32(0.5)
            z2 = jnp.float32(_TWO_PI_SQ) * sv * sv
            cacc = jnp.full((L,), jnp.float32(-1.0 / 87178291200.0))
            for coef in _COS_COEFS:
                cacc = cacc * z2 + jnp.float32(coef)
            cutv = jnp.float32(0.5) - jnp.float32(0.5) * cacc
            cut = cutv * cutv
            spv = spf.astype(jnp.int32)
            widx0 = spv * NWAVE + wbase
            fvals = []
            for wl in range(WH):
                widx = widx0 + wl
                rsv = plsc.load_gather(rs_t, [widx])
                itv = plsc.load_gather(inta_t, [widx])
                pv = plsc.load_gather(par_t, [widx])
                dd = dist - rsv
                fvals.append(cut * jnp.exp(-(itv * dd * dd)) * pv)
            dv = [dx, dy, dz]
            o9 = [dv[j] * dv[k] for j in range(3) for k in range(3)]
            ang = [None] + dv + o9 + [o9[m] * dv[k] for m in range(9) for k in range(3)]
            for p in range(P):
                for wl in range(WH):
                    val = fvals[wl] if p == 0 else ang[p] * fvals[wl]
                    colv = jnp.full((L,), p * WH + wl, jnp.int32)
                    plsc.store_scatter(radb, [rowv, colv], val)
        pltpu.sync_copy(radb, acc.at[dstb], add=True)
        return carry

    lax.fori_loop(0, NCHUNK, chunk_body, 0)
    plsc.subcore_barrier()
    pltpu.sync_copy(acc.at[pl.ds(s * NPT, NPT)],
                orb_h.at[pl.ds(c * N_PAD + s * NPT, NPT)])


def _sc_orbital(ctab, dst, src, shx, shy, shz, rs_f, inta_f, par_f, zeros):
    mesh = plsc.VectorSubcoreMesh(core_axis_name="c", subcore_axis_name="s",
                                  num_cores=NC, num_subcores=NS)
    f32, i32 = jnp.float32, jnp.int32
    kern = pl.kernel(
        _sc_body,
        out_type=jax.ShapeDtypeStruct((NC * N_PAD, F), f32),
        mesh=mesh,
        compiler_params=pltpu.CompilerParams(needs_layout_passes=False,
                                             use_tc_tiling_on_sc=False),
        scratch_types=[
            pltpu.VMEM((32,), f32),
            pltpu.VMEM((32,), f32),
            pltpu.VMEM((32,), f32),
            pltpu.VMEM((CHUNK,), i32),
            pltpu.VMEM((CHUNK,), i32),
            pltpu.VMEM((CHUNK,), f32),
            pltpu.VMEM((CHUNK,), f32),
            pltpu.VMEM((CHUNK,), f32),
            pltpu.VMEM((CHUNK, 16), f32),
            pltpu.VMEM((CHUNK, 16), f32),
            pltpu.VMEM((CHUNK, F), f32),
            pltpu.VMEM_SHARED((N_PAD, F), f32),
        ],
    )
    return kern(ctab, dst, src, shx, shy, shz, rs_f, inta_f, par_f, zeros)


def _density_tc(orb0, orb1, sel_m):
    rows = 2000

    def body(o0_ref, o1_ref, m_ref, out_ref):
        sq = jnp.concatenate([o0_ref[...], o1_ref[...]], axis=1)
        sq = sq * sq
        out_ref[...] = jnp.dot(sq, m_ref[...], preferred_element_type=jnp.float32)

    return pl.pallas_call(
        body,
        grid=(N_NODES // rows,),
        in_specs=[
            pl.BlockSpec((rows, F), lambda i: (i, 0)),
            pl.BlockSpec((rows, F), lambda i: (i, 0)),
            pl.BlockSpec((2 * F, 32), lambda i: (0, 0)),
        ],
        out_specs=pl.BlockSpec((rows, 32), lambda i: (i, 0)),
        out_shape=jax.ShapeDtypeStruct((N_NODES, 32), jnp.float32),
    )(orb0, orb1, sel_m)


def kernel(cart, neigh_list, shifts, species, rs, inta, params):
    f32, i32 = jnp.float32, jnp.int32
    cart = cart.astype(f32)
    ctab = jnp.concatenate([cart, species.astype(f32)[:, None],
                        jnp.zeros((N_NODES, 12), f32)], axis=1)
    dst = neigh_list[0].astype(i32)
    src = neigh_list[1].astype(i32)
    shifts = shifts.astype(f32)
    shx = shifts[:, 0]
    shy = shifts[:, 1]
    shz = shifts[:, 2]
    rs_f = rs.astype(f32).reshape(-1)
    inta_f = inta.astype(f32).reshape(-1)
    par_f = params.astype(f32).reshape(-1)
    zeros = jnp.zeros((N_PAD, F), f32)
    orb = _sc_orbital(ctab, dst, src, shx, shy, shz, rs_f, inta_f, par_f, zeros)
    return _density_tc(orb[:N_NODES], orb[N_PAD:N_PAD + N_NODES], jnp.asarray(_SEL_M_NP))


# speedup vs baseline: 21.0433x; 1.3857x over previous
import numpy as np
import jax
import jax.numpy as jnp
from jax import lax
from jax.experimental import pallas as pl
from jax.experimental.pallas import tpu as pltpu
from jax.experimental.pallas import tpu_sc as plsc

N_NODES = 10000
N_EDGES = 160000
NWAVE = 8
P = 40
NC, NS, L = 2, 16, 16
WH = NWAVE // NC
F = P * WH
CHUNK = 64
GPC = CHUNK // L
NCHUNK = 158
EPT = NCHUNK * CHUNK
E_PAD = EPT * NS
N_PAD = 10240
NPT = N_PAD // NS
DUMMY = N_PAD - 1

_TWO_PI_SQ = float(4.0 * np.pi * np.pi)
_COS_COEFS = [1.0 / 479001600.0, -1.0 / 3628800.0, 1.0 / 40320.0,
              -1.0 / 720.0, 1.0 / 24.0, -0.5, 1.0]

_INDEX_PARA = np.repeat(np.arange(4), [1, 3, 9, 27])


def _build_sel_matrix():
    m = np.zeros((2 * F, 32), np.float32)
    for j in range(2 * F):
        c, jj = divmod(j, F)
        p, wl = divmod(jj, WH)
        m[j, _INDEX_PARA[p] * NWAVE + c * WH + wl] = 1.0
    return m


_SEL_M_NP = _build_sel_matrix()


def _sc_body(ctab_h, dst_h, src_h, shx_h, shy_h, shz_h,
             rs_h, inta_h, par_h, zero_h, orb_h,
             rs_t, inta_t, par_t,
             dstb0, srcb0, shxb0, shyb0, shzb0, crowsd0, crowss0, radb0, scix0,
             dstb1, srcb1, shxb1, shyb1, shzb1, crowsd1, crowss1, radb1, scix1,
             angbuf, acc, sst0, sst1, sga0, sga1, ssc0, ssc1):
    c = lax.axis_index("c")
    s = lax.axis_index("s")

    bufs = (
        (dstb0, srcb0, shxb0, shyb0, shzb0, crowsd0, crowss0, radb0, scix0,
         sst0, sga0, ssc0),
        (dstb1, srcb1, shxb1, shyb1, shzb1, crowsd1, crowss1, radb1, scix1,
         sst1, sga1, ssc1),
    )

    pltpu.sync_copy(rs_h, rs_t)
    pltpu.sync_copy(inta_h, inta_t)
    pltpu.sync_copy(par_h, par_t)

    pltpu.sync_copy(zero_h.at[pl.ds(s * NPT, NPT)], acc.at[pl.ds(s * NPT, NPT)])
    plsc.subcore_barrier()

    wbase = c * WH
    iota = lax.iota(jnp.int32, L)
    czero = jnp.full((L,), 0, jnp.int32)
    cone = jnp.full((L,), 1, jnp.int32)
    ctwo = jnp.full((L,), 2, jnp.int32)
    cthree = jnp.full((L,), 3, jnp.int32)

    def stage_refs(ci, B):
        dstb, srcb, shxb, shyb, shzb = B[0], B[1], B[2], B[3], B[4]
        ebase = s * EPT + ci * CHUNK
        sl = pl.ds(ebase, CHUNK)
        return ((dst_h.at[sl], dstb), (src_h.at[sl], srcb),
                (shx_h.at[sl], shxb), (shy_h.at[sl], shyb),
                (shz_h.at[sl], shzb))

    def stage_start(ci, B):
        for src_ref, dst_ref in stage_refs(ci, B):
            pltpu.async_copy(src_ref, dst_ref, B[9])

    def stage_wait(ci, B):
        for src_ref, dst_ref in stage_refs(ci, B):
            pltpu.make_async_copy(src_ref, dst_ref, B[9]).wait()

    def gather_start(B):
        pltpu.async_copy(ctab_h.at[B[0]], B[5], B[10])
        pltpu.async_copy(ctab_h.at[B[1]], B[6], B[10])

    def gather_wait(B):
        pltpu.make_async_copy(ctab_h.at[B[0]], B[5], B[10]).wait()
        pltpu.make_async_copy(ctab_h.at[B[1]], B[6], B[10]).wait()

    def scatter_start(B):
        pltpu.async_copy(B[7], acc.at[B[8]], B[11], add=True)

    def scatter_wait(B):
        pltpu.make_async_copy(B[7], acc.at[B[8]], B[11]).wait()

    def compute_group(B, g):
        dstb, srcb, shxb, shyb, shzb, crowsd, crowss, radb = B[:8]
        rowv = iota + (g * L)
        xi = plsc.load_gather(crowsd, [rowv, czero])
        yi = plsc.load_gather(crowsd, [rowv, cone])
        zi = plsc.load_gather(crowsd, [rowv, ctwo])
        xj = plsc.load_gather(crowss, [rowv, czero])
        yj = plsc.load_gather(crowss, [rowv, cone])
        zj = plsc.load_gather(crowss, [rowv, ctwo])
        spf = plsc.load_gather(crowss, [rowv, cthree])
        dx = xi - xj - shxb[pl.ds(g * L, L)]
        dy = yi - yj - shyb[pl.ds(g * L, L)]
        dz = zi - zj - shzb[pl.ds(g * L, L)]
        r2 = dx * dx + dy * dy + dz * dz
        bits = plsc.bitcast(r2, jnp.int32)
        bits = jnp.int32(0x5F3759DF) - (bits >> 1)
        yv = plsc.bitcast(bits, jnp.float32)
        for _ in range(3):
            yv = yv * (jnp.float32(1.5) - jnp.float32(0.5) * r2 * yv * yv)
        dist = r2 * yv
        t = dist * jnp.float32(0.1)
        frac = t - t.astype(jnp.int32).astype(jnp.float32)
        sv = frac - jnp.float32(0.5)
        z2 = jnp.float32(_TWO_PI_SQ) * sv * sv
        cacc = jnp.full((L,), jnp.float32(-1.0 / 87178291200.0))
        for coef in _COS_COEFS:
            cacc = cacc * z2 + jnp.float32(coef)
        cutv = jnp.float32(0.5) - jnp.float32(0.5) * cacc
        cut = cutv * cutv
        spv = spf.astype(jnp.int32)
        widx0 = spv * NWAVE + wbase
        fvals = []
        for wl in range(WH):
            widx = widx0 + wl
            rsv = plsc.load_gather(rs_t, [widx])
            itv = plsc.load_gather(inta_t, [widx])
            pv = plsc.load_gather(par_t, [widx])
            dd = dist - rsv
            fvals.append(cut * jnp.exp(-(itv * dd * dd)) * pv)
        angbuf[pl.ds(0, L)] = jnp.full((L,), jnp.float32(1.0))
        dv = [dx, dy, dz]
        for j in range(3):
            angbuf[pl.ds((1 + j) * L, L)] = dv[j]
        o9 = []
        for j in range(3):
            for k in range(3):
                v = dv[j] * dv[k]
                o9.append(v)
                angbuf[pl.ds((4 + 3 * j + k) * L, L)] = v
        for m in range(9):
            for k in range(3):
                angbuf[pl.ds((13 + 3 * m + k) * L, L)] = o9[m] * dv[k]

        def pbody(pp, carry):
            for pu in range(4):
                p = pp * 4 + pu
                angv = angbuf[pl.ds(p * L, L)]
                colv0 = jnp.full((L,), 0, jnp.int32) + p * WH
                for wl in range(WH):
                    plsc.store_scatter(radb, [rowv, colv0 + wl],
                                       angv * fvals[wl])
            return carry

        lax.fori_loop(0, P // 4, pbody, 0)

    def process(ci, b):
        B, NB = bufs[b], bufs[1 - b]

        @pl.when(ci >= 2)
        def _():
            scatter_wait(B)

        @pl.when(ci + 1 < NCHUNK)
        def _():
            stage_start(ci + 1, NB)

        gather_wait(B)
        compute_group(B, 0)

        @pl.when(ci + 1 < NCHUNK)
        def _():
            stage_wait(ci + 1, NB)
            gather_start(NB)

        for g in range(1, GPC):
            compute_group(B, g)
        for q in range(GPC):
            B[8][pl.ds(q * L, L)] = B[0][pl.ds(q * L, L)]
        scatter_start(B)

    stage_start(0, bufs[0])
    stage_wait(0, bufs[0])
    gather_start(bufs[0])

    def pair_body(k, carry):
        process(2 * k, 0)
        process(2 * k + 1, 1)
        return carry

    lax.fori_loop(0, NCHUNK // 2, pair_body, 0)
    scatter_wait(bufs[0])
    scatter_wait(bufs[1])
    plsc.subcore_barrier()
    pltpu.sync_copy(acc.at[pl.ds(s * NPT, NPT)],
                    orb_h.at[pl.ds(c * N_PAD + s * NPT, NPT)])


def _sc_orbital(ctab, dst, src, shx, shy, shz, rs_f, inta_f, par_f, zeros):
    mesh = plsc.VectorSubcoreMesh(core_axis_name="c", subcore_axis_name="s",
                                  num_cores=NC, num_subcores=NS)
    f32, i32 = jnp.float32, jnp.int32
    buf_set = [
        pltpu.VMEM((CHUNK,), i32),
        pltpu.VMEM((CHUNK,), i32),
        pltpu.VMEM((CHUNK,), f32),
        pltpu.VMEM((CHUNK,), f32),
        pltpu.VMEM((CHUNK,), f32),
        pltpu.VMEM((CHUNK, 16), f32),
        pltpu.VMEM((CHUNK, 16), f32),
        pltpu.VMEM((CHUNK, F), f32),
        pltpu.VMEM((CHUNK,), i32),
    ]
    kern = pl.kernel(
        _sc_body,
        out_type=jax.ShapeDtypeStruct((NC * N_PAD, F), f32),
        mesh=mesh,
        compiler_params=pltpu.CompilerParams(needs_layout_passes=False,
                                             use_tc_tiling_on_sc=False),
        scratch_types=(
            [pltpu.VMEM((32,), f32)] * 3
            + buf_set + buf_set
            + [pltpu.VMEM((P * L,), jnp.float32)]
            + [pltpu.VMEM_SHARED((N_PAD, F), f32)]
            + [pltpu.SemaphoreType.DMA] * 6
        ),
    )
    return kern(ctab, dst, src, shx, shy, shz, rs_f, inta_f, par_f, zeros)


def _density_tc(orb0, orb1, sel_m):
    rows = 2000

    def body(o0_ref, o1_ref, m_ref, out_ref):
        sq = jnp.concatenate([o0_ref[...], o1_ref[...]], axis=1)
        sq = sq * sq
        out_ref[...] = jnp.dot(sq, m_ref[...], preferred_element_type=jnp.float32)

    return pl.pallas_call(
        body,
        grid=(N_NODES // rows,),
        in_specs=[
            pl.BlockSpec((rows, F), lambda i: (i, 0)),
            pl.BlockSpec((rows, F), lambda i: (i, 0)),
            pl.BlockSpec((2 * F, 32), lambda i: (0, 0)),
        ],
        out_specs=pl.BlockSpec((rows, 32), lambda i: (i, 0)),
        out_shape=jax.ShapeDtypeStruct((N_NODES, 32), jnp.float32),
    )(orb0, orb1, sel_m)


def kernel(cart, neigh_list, shifts, species, rs, inta, params):
    f32, i32 = jnp.float32, jnp.int32
    cart = cart.astype(f32)
    shifts = shifts.astype(f32)
    npad = E_PAD - N_EDGES
    ctab = jnp.concatenate(
        [cart, species.astype(f32)[:, None], jnp.zeros((N_NODES, 12), f32)],
        axis=1)
    ctab = jnp.concatenate([ctab, jnp.zeros((N_PAD - N_NODES, 16), f32)], axis=0)
    dst = jnp.concatenate([neigh_list[0].astype(i32),
                           jnp.full((npad,), DUMMY, i32)])
    src = jnp.concatenate([neigh_list[1].astype(i32), jnp.zeros((npad,), i32)])
    zpad = jnp.zeros((npad,), f32)
    shx = jnp.concatenate([shifts[:, 0], zpad])
    shy = jnp.concatenate([shifts[:, 1], zpad])
    shz = jnp.concatenate([shifts[:, 2], zpad])
    rs_f = rs.astype(f32).reshape(-1)
    inta_f = inta.astype(f32).reshape(-1)
    par_f = params.astype(f32).reshape(-1)
    zeros = jnp.zeros((N_PAD, F), f32)
    orb = _sc_orbital(ctab, dst, src, shx, shy, shz, rs_f, inta_f, par_f, zeros)
    return _density_tc(orb[:N_NODES], orb[N_PAD:N_PAD + N_NODES],
                       jnp.asarray(_SEL_M_NP))
